# baseline (device time: 99142 ns/iter reference)
import jax
import jax.numpy as jnp
from jax import lax
from jax.experimental import pallas as pl
from jax.experimental.pallas import tpu as pltpu

N_Y = 4
V_LOC = 8192
T = 1024
D = 1024
CHUNK = T // N_Y
N_STEPS = 2 * (N_Y - 1)
N_PANELS = 8
P_ROWS = V_LOC // N_PANELS


def kernel(ids, E):
    y = lax.axis_index("y")
    local = ids - y * V_LOC
    valid = jnp.logical_and(local >= 0, local < V_LOC)
    locids = jnp.where(valid, local, -1).astype(jnp.int32)
    onehot = jax.nn.one_hot(locids, V_LOC, dtype=jnp.bfloat16)
    return _fused_embed_allreduce(onehot, E)


def _fused_embed_allreduce(onehot, E):
    def body(
        oh_ref, e_ref, out_ref,
        ebf_ref, panel_ref, acc_ref, comm_ref,
        panel_sems, send_sems, recv_sems,
    ):
        my_x = lax.axis_index("x")
        my_y = lax.axis_index("y")
        my_z = lax.axis_index("z")
        right = (my_y + 1) % N_Y
        left = (my_y + N_Y - 1) % N_Y

        barrier_sem = pltpu.get_barrier_semaphore()
        for nbr in (left, right):
            pl.semaphore_signal(
                barrier_sem, inc=1,
                device_id=(my_x, nbr, my_z),
                device_id_type=pl.DeviceIdType.MESH,
            )
        pl.semaphore_wait(barrier_sem, 2)

        def panel_copy(p):
            return pltpu.make_async_copy(
                e_ref.at[pl.ds(p * P_ROWS, P_ROWS), :],
                panel_ref.at[p % 2],
                panel_sems.at[p % 2],
            )

        panel_copy(0).start()
        for p in range(N_PANELS):
            if p + 1 < N_PANELS:
                panel_copy(p + 1).start()
            panel_copy(p).wait()
            ebf_ref[pl.ds(p * P_ROWS, P_ROWS), :] = panel_ref[p % 2].astype(
                jnp.bfloat16
            )

        part = jax.lax.dot_general(
            oh_ref[...], ebf_ref[...],
            dimension_numbers=(((1,), (0,)), ((), ())),
            preferred_element_type=jnp.float32,
        )
        acc_ref[...] = part.astype(jnp.bfloat16)

        for s in range(N_Y - 1):
            send_c = ((my_y + N_Y - s) % N_Y) * CHUNK
            recv_chunk = (my_y + N_Y - s - 1) % N_Y
            rdma = pltpu.make_async_remote_copy(
                src_ref=acc_ref.at[pl.ds(send_c, CHUNK), :],
                dst_ref=comm_ref.at[s],
                send_sem=send_sems.at[s],
                recv_sem=recv_sems.at[s],
                device_id=(my_x, right, my_z),
                device_id_type=pl.DeviceIdType.MESH,
            )
            rdma.start()
            rdma.wait()
            acc_ref[pl.ds(recv_chunk * CHUNK, CHUNK), :] += comm_ref[s]

        for s in range(N_Y - 1):
            send_c = ((my_y + 1 + N_Y - s) % N_Y) * CHUNK
            recv_c = ((my_y + N_Y - s) % N_Y) * CHUNK
            slot = (N_Y - 1) + s
            rdma = pltpu.make_async_remote_copy(
                src_ref=acc_ref.at[pl.ds(send_c, CHUNK), :],
                dst_ref=comm_ref.at[slot],
                send_sem=send_sems.at[slot],
                recv_sem=recv_sems.at[slot],
                device_id=(my_x, right, my_z),
                device_id_type=pl.DeviceIdType.MESH,
            )
            rdma.start()
            out_ref[pl.ds(send_c, CHUNK), :] = acc_ref[
                pl.ds(send_c, CHUNK), :
            ].astype(jnp.float32)
            rdma.wait()
            acc_ref[pl.ds(recv_c, CHUNK), :] = comm_ref[slot]

        last_c = ((my_y + 2) % N_Y) * CHUNK
        out_ref[pl.ds(last_c, CHUNK), :] = acc_ref[
            pl.ds(last_c, CHUNK), :
        ].astype(jnp.float32)

    return pl.pallas_call(
        body,
        out_shape=jax.ShapeDtypeStruct((T, D), jnp.float32),
        in_specs=[
            pl.BlockSpec(memory_space=pltpu.VMEM),
            pl.BlockSpec(memory_space=pl.ANY),
        ],
        out_specs=pl.BlockSpec(memory_space=pltpu.VMEM),
        scratch_shapes=[
            pltpu.VMEM((V_LOC, D), jnp.bfloat16),
            pltpu.VMEM((2, P_ROWS, D), jnp.float32),
            pltpu.VMEM((T, D), jnp.bfloat16),
            pltpu.VMEM((N_STEPS, CHUNK, D), jnp.bfloat16),
            pltpu.SemaphoreType.DMA((2,)),
            pltpu.SemaphoreType.DMA((N_STEPS,)),
            pltpu.SemaphoreType.DMA((N_STEPS,)),
        ],
        compiler_params=pltpu.CompilerParams(
            collective_id=0,
            vmem_limit_bytes=63 * 1024 * 1024,
        ),
    )(onehot, E)


# device time: 93394 ns/iter; 1.0615x vs baseline; 1.0615x over previous
import jax
import jax.numpy as jnp
from jax import lax
from jax.experimental import pallas as pl
from jax.experimental.pallas import tpu as pltpu

N_Y = 4
V_LOC = 8192
T = 1024
D = 1024
CHUNK = T // N_Y
N_STEPS = 2 * (N_Y - 1)
N_PANELS = 8
P_ROWS = V_LOC // N_PANELS


def kernel(ids, E):
    y = lax.axis_index("y")
    local = ids - y * V_LOC
    valid = jnp.logical_and(local >= 0, local < V_LOC)
    locids = jnp.where(valid, local, -1).astype(jnp.int32)
    onehot = jax.nn.one_hot(locids, V_LOC, dtype=jnp.bfloat16)
    return _fused_embed_allreduce(onehot, E)


def _fused_embed_allreduce(onehot, E):
    def body(
        oh_ref, e_ref, out_ref,
        panel_ref, acc_ref, comm_ref,
        panel_sems, send_sems, recv_sems,
    ):
        my_x = lax.axis_index("x")
        my_y = lax.axis_index("y")
        my_z = lax.axis_index("z")
        right = (my_y + 1) % N_Y
        left = (my_y + N_Y - 1) % N_Y

        barrier_sem = pltpu.get_barrier_semaphore()
        for nbr in (left, right):
            pl.semaphore_signal(
                barrier_sem, inc=1,
                device_id=(my_x, nbr, my_z),
                device_id_type=pl.DeviceIdType.MESH,
            )
        pl.semaphore_wait(barrier_sem, 2)

        def panel_copy(p):
            return pltpu.make_async_copy(
                e_ref.at[pl.ds(p * P_ROWS, P_ROWS), :],
                panel_ref.at[p % 2],
                panel_sems.at[p % 2],
            )

        panel_copy(0).start()
        part = jnp.zeros((T, D), jnp.float32)
        for p in range(N_PANELS):
            if p + 1 < N_PANELS:
                panel_copy(p + 1).start()
            panel_copy(p).wait()
            part = part + jax.lax.dot_general(
                oh_ref[:, pl.ds(p * P_ROWS, P_ROWS)],
                panel_ref[p % 2].astype(jnp.bfloat16),
                dimension_numbers=(((1,), (0,)), ((), ())),
                preferred_element_type=jnp.float32,
            )
        acc_ref[...] = part.astype(jnp.bfloat16)

        for s in range(N_Y - 1):
            send_c = ((my_y + N_Y - s) % N_Y) * CHUNK
            recv_chunk = (my_y + N_Y - s - 1) % N_Y
            rdma = pltpu.make_async_remote_copy(
                src_ref=acc_ref.at[pl.ds(send_c, CHUNK), :],
                dst_ref=comm_ref.at[s],
                send_sem=send_sems.at[s],
                recv_sem=recv_sems.at[s],
                device_id=(my_x, right, my_z),
                device_id_type=pl.DeviceIdType.MESH,
            )
            rdma.start()
            rdma.wait()
            acc_ref[pl.ds(recv_chunk * CHUNK, CHUNK), :] += comm_ref[s]

        for s in range(N_Y - 1):
            send_c = ((my_y + 1 + N_Y - s) % N_Y) * CHUNK
            recv_c = ((my_y + N_Y - s) % N_Y) * CHUNK
            slot = (N_Y - 1) + s
            rdma = pltpu.make_async_remote_copy(
                src_ref=acc_ref.at[pl.ds(send_c, CHUNK), :],
                dst_ref=comm_ref.at[slot],
                send_sem=send_sems.at[slot],
                recv_sem=recv_sems.at[slot],
                device_id=(my_x, right, my_z),
                device_id_type=pl.DeviceIdType.MESH,
            )
            rdma.start()
            out_ref[pl.ds(send_c, CHUNK), :] = acc_ref[
                pl.ds(send_c, CHUNK), :
            ].astype(jnp.float32)
            rdma.wait()
            acc_ref[pl.ds(recv_c, CHUNK), :] = comm_ref[slot]

        last_c = ((my_y + 2) % N_Y) * CHUNK
        out_ref[pl.ds(last_c, CHUNK), :] = acc_ref[
            pl.ds(last_c, CHUNK), :
        ].astype(jnp.float32)

    return pl.pallas_call(
        body,
        out_shape=jax.ShapeDtypeStruct((T, D), jnp.float32),
        in_specs=[
            pl.BlockSpec(memory_space=pltpu.VMEM),
            pl.BlockSpec(memory_space=pl.ANY),
        ],
        out_specs=pl.BlockSpec(memory_space=pltpu.VMEM),
        scratch_shapes=[
            pltpu.VMEM((2, P_ROWS, D), jnp.float32),
            pltpu.VMEM((T, D), jnp.bfloat16),
            pltpu.VMEM((N_STEPS, CHUNK, D), jnp.bfloat16),
            pltpu.SemaphoreType.DMA((2,)),
            pltpu.SemaphoreType.DMA((N_STEPS,)),
            pltpu.SemaphoreType.DMA((N_STEPS,)),
        ],
        compiler_params=pltpu.CompilerParams(
            collective_id=0,
            vmem_limit_bytes=63 * 1024 * 1024,
        ),
    )(onehot, E)
